# baseline (speedup 1.0000x reference)
import functools

import jax
import jax.numpy as jnp
from jax import lax
from jax.experimental import pallas as pl
from jax.experimental.pallas import tpu as pltpu
from jax.experimental.pallas import tpu_sc as plsc

B, C, T, H, W = 8, 384, 32, 16, 16
N = B * C * T
HW = H * W
NW = 32
ROWS_PER_W = N // NW
CHUNK = 128
NCHUNK = ROWS_PER_W // CHUNK
GROUPS = CHUNK // 16
SCALE = 1.0 / 210.0


def _sc_call(xf):
    mesh = plsc.VectorSubcoreMesh(core_axis_name="c", subcore_axis_name="s")

    @functools.partial(
        pl.kernel,
        mesh=mesh,
        out_type=jax.ShapeDtypeStruct((N,), jnp.float32),
        compiler_params=pltpu.CompilerParams(needs_layout_passes=False),
        scratch_types=[
            pltpu.VMEM((CHUNK, HW), jnp.float32),
            pltpu.VMEM((CHUNK,), jnp.float32),
        ],
    )
    def k(x_hbm, out_hbm, buf, obuf):
        cid = lax.axis_index("c")
        sid = lax.axis_index("s")
        wid = sid * 2 + cid
        base = wid * ROWS_PER_W
        lanes = lax.iota(jnp.int32, 16)

        def chunk_body(ci, carry):
            row0 = base + ci * CHUNK
            pltpu.sync_copy(x_hbm.at[pl.ds(row0, CHUNK)], buf)

            def group_body(g, carry):
                rows = g * 16 + lanes

                def r_body(r, accs):
                    a0, a1 = accs
                    p0 = r * 16
                    for cc in range(1, 15):
                        v = plsc.load_gather(buf, [rows, jnp.full((16,), 0, jnp.int32) + (p0 + cc)])
                        if cc % 2 == 0:
                            a0 = a0 + v
                        else:
                            a1 = a1 + v
                    return a0, a1

                zero = jnp.zeros((16,), jnp.float32)
                a0, a1 = lax.fori_loop(0, 15, r_body, (zero, zero))
                obuf[pl.ds(g * 16, 16)] = (a0 + a1) * SCALE
                return carry

            lax.fori_loop(0, GROUPS, group_body, 0)
            pltpu.sync_copy(obuf, out_hbm.at[pl.ds(row0, CHUNK)])
            return carry

        lax.fori_loop(0, NCHUNK, chunk_body, 0)

    return k(xf)


def kernel(x):
    b, c, t, h, w = x.shape
    xf = x.reshape(b * c * t, h * w)
    out = _sc_call(xf)
    return out.reshape(b, c, t)


# speedup vs baseline: 1.3447x; 1.3447x over previous
import functools

import jax
import jax.numpy as jnp
from jax import lax
from jax.experimental import pallas as pl
from jax.experimental.pallas import tpu as pltpu
from jax.experimental.pallas import tpu_sc as plsc

B, C, T, H, W = 8, 384, 32, 16, 16
N = B * C * T
HW = H * W
NW = 32
ROWS_PER_W = N // NW
CHUNK = 128
NCHUNK = ROWS_PER_W // CHUNK
GROUPS = CHUNK // 16
SCALE = 1.0 / 210.0


def _sc_call(xf):
    mesh = plsc.VectorSubcoreMesh(core_axis_name="c", subcore_axis_name="s")

    @functools.partial(
        pl.kernel,
        mesh=mesh,
        out_type=jax.ShapeDtypeStruct((N,), jnp.float32),
        compiler_params=pltpu.CompilerParams(needs_layout_passes=False),
        scratch_types=[
            pltpu.VMEM((CHUNK, HW), jnp.float32),
            pltpu.VMEM((16, 17), jnp.float32),
            pltpu.VMEM((CHUNK,), jnp.float32),
        ],
    )
    def k(x_hbm, out_hbm, buf, scr, obuf):
        cid = lax.axis_index("c")
        sid = lax.axis_index("s")
        wid = sid * 2 + cid
        base = wid * ROWS_PER_W
        lanes = lax.iota(jnp.int32, 16)

        def chunk_body(ci, carry):
            row0 = base + ci * CHUNK
            pltpu.sync_copy(x_hbm.at[pl.ds(row0, CHUNK)], buf)

            def group_body(g, carry):
                def i_body(i, carry):
                    row = g * 16 + i
                    a0 = buf[row, pl.ds(0, 16)]
                    a1 = buf[row, pl.ds(16, 16)]
                    a2 = buf[row, pl.ds(32, 16)]
                    for r in range(3, 15):
                        v = buf[row, pl.ds(16 * r, 16)]
                        if r % 3 == 0:
                            a0 = a0 + v
                        elif r % 3 == 1:
                            a1 = a1 + v
                        else:
                            a2 = a2 + v
                    scr[i, pl.ds(0, 16)] = (a0 + a1) + a2
                    return carry

                lax.fori_loop(0, 16, i_body, 0)

                z = jnp.zeros((16,), jnp.float32)
                b0, b1 = z, z
                for cc in range(1, 15):
                    v = plsc.load_gather(scr, [lanes, jnp.full((16,), cc, jnp.int32)])
                    if cc % 2 == 0:
                        b0 = b0 + v
                    else:
                        b1 = b1 + v
                obuf[pl.ds(g * 16, 16)] = (b0 + b1) * SCALE
                return carry

            lax.fori_loop(0, GROUPS, group_body, 0)
            pltpu.sync_copy(obuf, out_hbm.at[pl.ds(row0, CHUNK)])
            return carry

        lax.fori_loop(0, NCHUNK, chunk_body, 0)

    return k(xf)


def kernel(x):
    b, c, t, h, w = x.shape
    xf = x.reshape(b * c * t, h * w)
    out = _sc_call(xf)
    return out.reshape(b, c, t)


# speedup vs baseline: 1.4800x; 1.1006x over previous
import functools

import jax
import jax.numpy as jnp
from jax import lax
from jax.experimental import pallas as pl
from jax.experimental.pallas import tpu as pltpu
from jax.experimental.pallas import tpu_sc as plsc

B, C, T, H, W = 8, 384, 32, 16, 16
N = B * C * T
HW = H * W
NW = 32
ROWS_PER_W = N // NW
CHUNK = 128
NCHUNK = ROWS_PER_W // CHUNK
GROUPS = CHUNK // 16
SCALE = 1.0 / 210.0


def _sc_call(xf):
    mesh = plsc.VectorSubcoreMesh(core_axis_name="c", subcore_axis_name="s")

    @functools.partial(
        pl.kernel,
        mesh=mesh,
        out_type=jax.ShapeDtypeStruct((N,), jnp.float32),
        compiler_params=pltpu.CompilerParams(needs_layout_passes=False),
        scratch_types=[
            pltpu.VMEM((CHUNK, HW), jnp.float32),
            pltpu.VMEM((CHUNK, HW), jnp.float32),
            pltpu.VMEM((CHUNK, 17), jnp.float32),
            pltpu.VMEM((CHUNK,), jnp.float32),
            pltpu.VMEM((CHUNK,), jnp.float32),
            pltpu.SemaphoreType.DMA,
            pltpu.SemaphoreType.DMA,
            pltpu.SemaphoreType.DMA,
            pltpu.SemaphoreType.DMA,
        ],
    )
    def k(x_hbm, out_hbm, buf0, buf1, scr, ob0, ob1, is0, is1, os0, os1):
        cid = lax.axis_index("c")
        sid = lax.axis_index("s")
        wid = sid * 2 + cid
        base = wid * ROWS_PER_W
        lanes = lax.iota(jnp.int32, 16)
        bufs = (buf0, buf1)
        obs = (ob0, ob1)
        isems = (is0, is1)
        osems = (os0, os1)

        def in_chunk(ci):
            return x_hbm.at[pl.ds(base + ci * CHUNK, CHUNK)]

        def out_chunk(ci):
            return out_hbm.at[pl.ds(base + ci * CHUNK, CHUNK)]

        pltpu.async_copy(in_chunk(0), buf0, is0)
        pltpu.async_copy(in_chunk(1), buf1, is1)

        def compute(buf, ob):
            @plsc.parallel_loop(0, CHUNK, unroll=4)
            def _(i):
                a0 = buf[i, pl.ds(0, 16)]
                a1 = buf[i, pl.ds(16, 16)]
                a2 = buf[i, pl.ds(32, 16)]
                for r in range(3, 15):
                    v = buf[i, pl.ds(16 * r, 16)]
                    if r % 3 == 0:
                        a0 = a0 + v
                    elif r % 3 == 1:
                        a1 = a1 + v
                    else:
                        a2 = a2 + v
                scr[i, pl.ds(0, 16)] = (a0 + a1) + a2

            @plsc.parallel_loop(0, GROUPS, unroll=2)
            def _(g):
                z = jnp.zeros((16,), jnp.float32)
                b0, b1 = z, z
                rows = g * 16 + lanes
                for cc in range(1, 15):
                    v = plsc.load_gather(
                        scr, [rows, jnp.full((16,), cc, jnp.int32)])
                    if cc % 2 == 0:
                        b0 = b0 + v
                    else:
                        b1 = b1 + v
                ob[pl.ds(g * 16, 16)] = (b0 + b1) * SCALE

        def pair_body(kk, carry):
            for b in range(2):
                ci = 2 * kk + b
                buf, ob = bufs[b], obs[b]
                pltpu.make_async_copy(in_chunk(ci), buf, isems[b]).wait()
                @pl.when(kk > 0)
                def _():
                    pltpu.make_async_copy(ob, out_chunk(ci), osems[b]).wait()
                compute(buf, ob)
                @pl.when(ci + 2 < NCHUNK)
                def _():
                    pltpu.async_copy(in_chunk(ci + 2), buf, isems[b])
                pltpu.async_copy(ob, out_chunk(ci), osems[b])
            return carry

        lax.fori_loop(0, NCHUNK // 2, pair_body, 0)

        for b in range(2):
            pltpu.make_async_copy(
                obs[b], out_chunk(NCHUNK - 2 + b), osems[b]).wait()

    return k(xf)


def kernel(x):
    b, c, t, h, w = x.shape
    xf = x.reshape(b * c * t, h * w)
    out = _sc_call(xf)
    return out.reshape(b, c, t)
